# Initial kernel scaffold; baseline (speedup 1.0000x reference)
import jax
import jax.numpy as jnp
from jax.experimental import pallas as pl


def kernel(hidden_states, idxes, Wq, bq, Wk, bk, Wv, bv, Wo, bo, ln1_g, ln1_b, fc1_W, fc1_b, fc2_W, fc2_b, exp1_W, exp1_b, exp2_W, gate_W, gate_b, fln_g, fln_b):
    raise NotImplementedError("write your pallas kernel here")


# speedup vs baseline: 3.1663x; 3.1663x over previous
import jax
import jax.numpy as jnp
from jax.experimental import pallas as pl
from jax.experimental.pallas import tpu as pltpu

B, S, D, H = 2, 2048, 768, 12
FFN, INTER, E, ND = 3072, 768, 8, 4
DH = D // H
T = B * S
SCALE = DH ** -0.5

TB_QKV = 512
BQ = 512
TB = 256
NTB = T // TB


def _gelu(x):
    return x * 0.5 * (1.0 + jax.lax.erf(x * (2.0 ** -0.5)))


def _layernorm(t, g, b):
    m = jnp.mean(t, axis=-1, keepdims=True)
    v = jnp.mean((t - m) ** 2, axis=-1, keepdims=True)
    return (t - m) / jnp.sqrt(v + 1e-5) * g + b


def _qkv_body(x_ref, w_ref, b_ref, o_ref):
    o_ref[...] = (jnp.dot(x_ref[...], w_ref[...],
                          preferred_element_type=jnp.float32) + b_ref[...])


def _attn_body(q_ref, k_ref, v_ref, o_ref):
    q = q_ref[0, 0]
    k = k_ref[0, 0]
    v = v_ref[0, 0]
    s = jax.lax.dot_general(q, k, (((1,), (1,)), ((), ())),
                            preferred_element_type=jnp.float32)
    m = jnp.max(s, axis=-1, keepdims=True)
    p = jnp.exp(s - m)
    p = p / jnp.sum(p, axis=-1, keepdims=True)
    o_ref[0, 0] = jnp.dot(p, v, preferred_element_type=jnp.float32)


def _oproj_ln_body(o_ref, w_ref, b_ref, hs_ref, g_ref, bb_ref, x_ref):
    t = (jnp.dot(o_ref[...], w_ref[...], preferred_element_type=jnp.float32)
         + b_ref[...] + hs_ref[...])
    x_ref[...] = _layernorm(t, g_ref[...], bb_ref[...])


def _gate_body(idx_ref, x_ref, gw_ref, gb_ref, gate_ref, gval_ref):
    del idx_ref
    x = x_ref[...]
    gw = gw_ref[0]
    logits = jax.lax.dot_general(x, gw, (((1,), (1,)), ((), ())),
                                 preferred_element_type=jnp.float32)
    logits = logits + gb_ref[0]
    m = jnp.max(logits, axis=-1, keepdims=True)
    p = jnp.exp(logits - m)
    sp = jnp.sum(p, axis=-1)
    gate_ref[0, 0] = jnp.argmax(logits, axis=-1).astype(jnp.int32)
    gval_ref[0, 0] = jnp.max(p, axis=-1) / sp


def _ffn_body(x_ref, w1_ref, b1_ref, w2_ref, b2_ref,
              e1w_ref, e1b_ref, e2w_ref, gate_ref, gval_ref,
              g_ref, bb_ref, o_ref, acc_ref):
    e = pl.program_id(1)
    x = x_ref[...]

    @pl.when(e == 0)
    def _init():
        h = jnp.dot(x, w1_ref[...], preferred_element_type=jnp.float32)
        h = _gelu(h + b1_ref[...])
        acc_ref[...] = (jnp.dot(h, w2_ref[...],
                                preferred_element_type=jnp.float32)
                        + b2_ref[...])

    he = jax.lax.dot_general(x, e1w_ref[0], (((1,), (1,)), ((), ())),
                             preferred_element_type=jnp.float32)
    he = _gelu(he + e1b_ref[0])
    ye = jax.lax.dot_general(he, e2w_ref[0], (((1,), (1,)), ((), ())),
                             preferred_element_type=jnp.float32)
    mask = (gate_ref[0, 0] == e).astype(jnp.float32)[:, None]
    acc_ref[...] += mask * ye

    @pl.when(e == E - 1)
    def _fin():
        t = acc_ref[...] * gval_ref[0, 0][:, None] + x
        o_ref[...] = _layernorm(t, g_ref[...], bb_ref[...])


def kernel(hidden_states, idxes, Wq, bq, Wk, bk, Wv, bv, Wo, bo, ln1_g, ln1_b,
           fc1_W, fc1_b, fc2_W, fc2_b, exp1_W, exp1_b, exp2_W, gate_W, gate_b,
           fln_g, fln_b):
    xf = hidden_states.reshape(T, D)
    Wqkv = jnp.concatenate([Wq.T * SCALE, Wk.T, Wv.T], axis=1)
    bqkv = jnp.concatenate([bq * SCALE, bk, bv]).reshape(1, 3 * D)

    qkv = pl.pallas_call(
        _qkv_body,
        grid=(T // TB_QKV,),
        in_specs=[pl.BlockSpec((TB_QKV, D), lambda i: (i, 0)),
                  pl.BlockSpec((D, 3 * D), lambda i: (0, 0)),
                  pl.BlockSpec((1, 3 * D), lambda i: (0, 0))],
        out_specs=pl.BlockSpec((TB_QKV, 3 * D), lambda i: (i, 0)),
        out_shape=jax.ShapeDtypeStruct((T, 3 * D), jnp.float32),
    )(xf, Wqkv, bqkv)

    q = qkv[:, :D].reshape(B, S, H, DH).transpose(0, 2, 1, 3)
    k = qkv[:, D:2 * D].reshape(B, S, H, DH).transpose(0, 2, 1, 3)
    v = qkv[:, 2 * D:].reshape(B, S, H, DH).transpose(0, 2, 1, 3)

    o = pl.pallas_call(
        _attn_body,
        grid=(B, H, S // BQ),
        in_specs=[pl.BlockSpec((1, 1, BQ, DH), lambda b, h, i: (b, h, i, 0)),
                  pl.BlockSpec((1, 1, S, DH), lambda b, h, i: (b, h, 0, 0)),
                  pl.BlockSpec((1, 1, S, DH), lambda b, h, i: (b, h, 0, 0))],
        out_specs=pl.BlockSpec((1, 1, BQ, DH), lambda b, h, i: (b, h, i, 0)),
        out_shape=jax.ShapeDtypeStruct((B, H, S, DH), jnp.float32),
    )(q, k, v)

    of = o.transpose(0, 2, 1, 3).reshape(T, D)

    x = pl.pallas_call(
        _oproj_ln_body,
        grid=(NTB,),
        in_specs=[pl.BlockSpec((TB, D), lambda i: (i, 0)),
                  pl.BlockSpec((D, D), lambda i: (0, 0)),
                  pl.BlockSpec((1, D), lambda i: (0, 0)),
                  pl.BlockSpec((TB, D), lambda i: (i, 0)),
                  pl.BlockSpec((1, D), lambda i: (0, 0)),
                  pl.BlockSpec((1, D), lambda i: (0, 0))],
        out_specs=pl.BlockSpec((TB, D), lambda i: (i, 0)),
        out_shape=jax.ShapeDtypeStruct((T, D), jnp.float32),
    )(of, Wo.T, bo.reshape(1, D), xf,
      ln1_g.reshape(1, D), ln1_b.reshape(1, D))

    grid_spec = pltpu.PrefetchScalarGridSpec(
        num_scalar_prefetch=1,
        grid=(B,),
        in_specs=[pl.BlockSpec((S, D), lambda b, idx: (b, 0)),
                  pl.BlockSpec((1, E, D), lambda b, idx: (idx[b], 0, 0)),
                  pl.BlockSpec((1, 1, E), lambda b, idx: (idx[b], 0, 0))],
        out_specs=[pl.BlockSpec((1, 1, S), lambda b, idx: (b, 0, 0)),
                   pl.BlockSpec((1, 1, S), lambda b, idx: (b, 0, 0))],
    )
    gate, gval = pl.pallas_call(
        _gate_body,
        grid_spec=grid_spec,
        out_shape=[jax.ShapeDtypeStruct((B, 1, S), jnp.int32),
                   jax.ShapeDtypeStruct((B, 1, S), jnp.float32)],
    )(idxes, x, gate_W, gate_b.reshape(ND, 1, E))

    gate_r = gate.reshape(NTB, 1, TB)
    gval_r = gval.reshape(NTB, 1, TB)

    out = pl.pallas_call(
        _ffn_body,
        grid=(NTB, E),
        in_specs=[pl.BlockSpec((TB, D), lambda i, e: (i, 0)),
                  pl.BlockSpec((D, FFN), lambda i, e: (0, 0)),
                  pl.BlockSpec((1, FFN), lambda i, e: (0, 0)),
                  pl.BlockSpec((FFN, D), lambda i, e: (0, 0)),
                  pl.BlockSpec((1, D), lambda i, e: (0, 0)),
                  pl.BlockSpec((1, INTER, D), lambda i, e: (e, 0, 0)),
                  pl.BlockSpec((1, 1, INTER), lambda i, e: (e, 0, 0)),
                  pl.BlockSpec((1, D, INTER), lambda i, e: (e, 0, 0)),
                  pl.BlockSpec((1, 1, TB), lambda i, e: (i, 0, 0)),
                  pl.BlockSpec((1, 1, TB), lambda i, e: (i, 0, 0)),
                  pl.BlockSpec((1, D), lambda i, e: (0, 0)),
                  pl.BlockSpec((1, D), lambda i, e: (0, 0))],
        out_specs=pl.BlockSpec((TB, D), lambda i, e: (i, 0)),
        out_shape=jax.ShapeDtypeStruct((T, D), jnp.float32),
        scratch_shapes=[pltpu.VMEM((TB, D), jnp.float32)],
    )(x, fc1_W.T, fc1_b.reshape(1, FFN), fc2_W.T, fc2_b.reshape(1, D),
      exp1_W, exp1_b.reshape(E, 1, INTER), exp2_W, gate_r, gval_r,
      fln_g.reshape(1, D), fln_b.reshape(1, D))

    return out.reshape(B, S, D)


# speedup vs baseline: 3.2552x; 1.0281x over previous
import functools

import jax
import jax.numpy as jnp
from jax.experimental import pallas as pl
from jax.experimental.pallas import tpu as pltpu
from jax.experimental.pallas import tpu_sc as plsc

B, S, D, H = 2, 2048, 768, 12
FFN, INTER, E, ND = 3072, 768, 8, 4
DH = D // H
T = B * S
SCALE = DH ** -0.5

TB_QKV = 512
BQ = 512
TB = 256
NTB = T // TB

TBS = 256
NB = 24
PAD_T = NB * TBS

TROWS = T // 128


def _gelu(x):
    return x * 0.5 * (1.0 + jax.lax.erf(x * (2.0 ** -0.5)))


def _layernorm(t, g, b):
    m = jnp.mean(t, axis=-1, keepdims=True)
    v = jnp.mean((t - m) ** 2, axis=-1, keepdims=True)
    return (t - m) / jnp.sqrt(v + 1e-5) * g + b


def _qkv_body(x_ref, w_ref, b_ref, o_ref):
    o_ref[...] = (jnp.dot(x_ref[...], w_ref[...],
                          preferred_element_type=jnp.float32) + b_ref[...])


def _attn_body(q_ref, k_ref, v_ref, o_ref):
    q = q_ref[0, 0]
    k = k_ref[0, 0]
    v = v_ref[0, 0]
    s = jax.lax.dot_general(q, k, (((1,), (1,)), ((), ())),
                            preferred_element_type=jnp.float32)
    m = jnp.max(s, axis=-1, keepdims=True)
    p = jnp.exp(s - m)
    p = p / jnp.sum(p, axis=-1, keepdims=True)
    o_ref[0, 0] = jnp.dot(p, v, preferred_element_type=jnp.float32)


def _oproj_ln_body(o_ref, w_ref, b_ref, hs_ref, g_ref, bb_ref, x_ref):
    t = (jnp.dot(o_ref[...], w_ref[...], preferred_element_type=jnp.float32)
         + b_ref[...] + hs_ref[...])
    x_ref[...] = _layernorm(t, g_ref[...], bb_ref[...])


def _gate_body(idx_ref, x_ref, gw_ref, gb_ref, gate_ref, gval_ref):
    del idx_ref
    x = x_ref[...]
    gw = gw_ref[0]
    logits = jax.lax.dot_general(x, gw, (((1,), (1,)), ((), ())),
                                 preferred_element_type=jnp.float32)
    logits = logits + gb_ref[0]
    m = jnp.max(logits, axis=-1, keepdims=True)
    p = jnp.exp(logits - m)
    sp = jnp.sum(p, axis=-1)
    gate_ref[0, 0] = jnp.argmax(logits, axis=-1).astype(jnp.int32)
    gval_ref[0, 0] = jnp.max(p, axis=-1) / sp


def _route_body(gate_ref, dst_ref, bex_ref, src_ref, dsts_ref):
    j = pl.program_id(0)

    @pl.when(j == 0)
    def _meta():
        g = gate_ref[...]
        l0 = jax.lax.broadcasted_iota(jnp.int32, (128, 128), 0)
        l1 = jax.lax.broadcasted_iota(jnp.int32, (128, 128), 1)
        slt_lane = (l0 < l1).astype(jnp.float32)
        r0 = jax.lax.broadcasted_iota(jnp.int32, (TROWS, TROWS), 0)
        r1 = jax.lax.broadcasted_iota(jnp.int32, (TROWS, TROWS), 1)
        slt_row = (r1 < r0).astype(jnp.float32)

        dst = jnp.zeros((TROWS, 128), jnp.float32)
        bpos = (jax.lax.broadcasted_iota(jnp.int32, (1, NB), 1)
                * TBS).astype(jnp.float32)
        bex = jnp.zeros((1, NB), jnp.float32)
        base = jnp.float32(0.0)
        for e in range(E):
            oh = (g == e).astype(jnp.float32)
            cs = jnp.dot(oh, slt_lane,
                         preferred_element_type=jnp.float32)
            rs = jnp.sum(oh, axis=1, keepdims=True)
            rp = jnp.dot(slt_row, rs,
                         preferred_element_type=jnp.float32)
            dst = dst + oh * (cs + rp + base)
            cnt = jnp.sum(oh)
            pc = jnp.ceil(cnt * (1.0 / TBS)) * TBS
            bex = bex + jnp.float32(e) * ((bpos >= base) &
                                          (bpos < base + pc)).astype(jnp.float32)
            base = base + pc
        dst_ref[...] = dst.astype(jnp.int32)
        dsts_ref[...] = dst.astype(jnp.int32)
        bex_ref[...] = bex.astype(jnp.int32)
        src_ref[0, 0] = jnp.zeros((TBS,), jnp.int32)

    @pl.when(j > 0)
    def _src():
        p0 = (j - 1) * TBS
        d = dsts_ref[...]
        pos = (jax.lax.broadcasted_iota(jnp.int32, (TBS, TROWS, 128), 0)
               + p0)
        eq = (d[None, :, :] == pos).astype(jnp.float32)
        tok = (jax.lax.broadcasted_iota(jnp.int32, (TBS, TROWS, 128), 1)
               * 128
               + jax.lax.broadcasted_iota(jnp.int32, (TBS, TROWS, 128), 2)
               ).astype(jnp.float32)
        s = jnp.sum(jnp.sum(eq * tok, axis=2), axis=1)
        src_ref[0, 0] = s.astype(jnp.int32)


def _expert_body(bex_ref, xs_ref, e1w_ref, e1b_ref, e2w_ref, ys_ref):
    del bex_ref
    xs = xs_ref[...]
    he = jax.lax.dot_general(xs, e1w_ref[0], (((1,), (1,)), ((), ())),
                             preferred_element_type=jnp.float32)
    he = _gelu(he + e1b_ref[0])
    ys_ref[...] = jax.lax.dot_general(he, e2w_ref[0], (((1,), (1,)), ((), ())),
                                      preferred_element_type=jnp.float32)


def _final_body(x_ref, w1_ref, b1_ref, w2_ref, b2_ref, ye_ref, gval_ref,
                g_ref, bb_ref, o_ref):
    x = x_ref[...]
    h = jnp.dot(x, w1_ref[...], preferred_element_type=jnp.float32)
    h = _gelu(h + b1_ref[...])
    y = (jnp.dot(h, w2_ref[...], preferred_element_type=jnp.float32)
         + b2_ref[...] + ye_ref[...])
    t = y * gval_ref[0, 0][:, None] + x
    o_ref[...] = _layernorm(t, g_ref[...], bb_ref[...])


def _sc_gather(table, idx, n_rows):
    info = plsc.get_sparse_core_info()
    nc, ns = info.num_cores, info.num_subcores
    nw = nc * ns
    per_w = n_rows // nw
    ch = 64
    n_ch = per_w // ch
    mesh = plsc.VectorSubcoreMesh(core_axis_name="c", subcore_axis_name="s")

    @functools.partial(
        pl.kernel, mesh=mesh,
        out_type=jax.ShapeDtypeStruct((n_rows, D), jnp.float32),
        scratch_types=[pltpu.VMEM((ch,), jnp.int32),
                       pltpu.VMEM((ch, D), jnp.float32),
                       pltpu.SemaphoreType.DMA])
    def gk(table_hbm, idx_hbm, out_hbm, idx_v, rows_v, sem):
        wid = jax.lax.axis_index("s") * nc + jax.lax.axis_index("c")
        base = wid * per_w
        for c in range(n_ch):
            off = base + c * ch
            pltpu.sync_copy(idx_hbm.at[pl.ds(off, ch)], idx_v)
            pltpu.async_copy(table_hbm.at[idx_v], rows_v, sem).wait()
            pltpu.sync_copy(rows_v, out_hbm.at[pl.ds(off, ch)])

    return gk(table, idx)


def kernel(hidden_states, idxes, Wq, bq, Wk, bk, Wv, bv, Wo, bo, ln1_g, ln1_b,
           fc1_W, fc1_b, fc2_W, fc2_b, exp1_W, exp1_b, exp2_W, gate_W, gate_b,
           fln_g, fln_b):
    xf = hidden_states.reshape(T, D)
    Wqkv = jnp.concatenate([Wq.T * SCALE, Wk.T, Wv.T], axis=1)
    bqkv = jnp.concatenate([bq * SCALE, bk, bv]).reshape(1, 3 * D)

    qkv = pl.pallas_call(
        _qkv_body,
        grid=(T // TB_QKV,),
        in_specs=[pl.BlockSpec((TB_QKV, D), lambda i: (i, 0)),
                  pl.BlockSpec((D, 3 * D), lambda i: (0, 0)),
                  pl.BlockSpec((1, 3 * D), lambda i: (0, 0))],
        out_specs=pl.BlockSpec((TB_QKV, 3 * D), lambda i: (i, 0)),
        out_shape=jax.ShapeDtypeStruct((T, 3 * D), jnp.float32),
    )(xf, Wqkv, bqkv)

    q = qkv[:, :D].reshape(B, S, H, DH).transpose(0, 2, 1, 3)
    k = qkv[:, D:2 * D].reshape(B, S, H, DH).transpose(0, 2, 1, 3)
    v = qkv[:, 2 * D:].reshape(B, S, H, DH).transpose(0, 2, 1, 3)

    o = pl.pallas_call(
        _attn_body,
        grid=(B, H, S // BQ),
        in_specs=[pl.BlockSpec((1, 1, BQ, DH), lambda b, h, i: (b, h, i, 0)),
                  pl.BlockSpec((1, 1, S, DH), lambda b, h, i: (b, h, 0, 0)),
                  pl.BlockSpec((1, 1, S, DH), lambda b, h, i: (b, h, 0, 0))],
        out_specs=pl.BlockSpec((1, 1, BQ, DH), lambda b, h, i: (b, h, i, 0)),
        out_shape=jax.ShapeDtypeStruct((B, H, S, DH), jnp.float32),
    )(q, k, v)

    of = o.transpose(0, 2, 1, 3).reshape(T, D)

    x = pl.pallas_call(
        _oproj_ln_body,
        grid=(NTB,),
        in_specs=[pl.BlockSpec((TB, D), lambda i: (i, 0)),
                  pl.BlockSpec((D, D), lambda i: (0, 0)),
                  pl.BlockSpec((1, D), lambda i: (0, 0)),
                  pl.BlockSpec((TB, D), lambda i: (i, 0)),
                  pl.BlockSpec((1, D), lambda i: (0, 0)),
                  pl.BlockSpec((1, D), lambda i: (0, 0))],
        out_specs=pl.BlockSpec((TB, D), lambda i: (i, 0)),
        out_shape=jax.ShapeDtypeStruct((T, D), jnp.float32),
    )(of, Wo.T, bo.reshape(1, D), xf,
      ln1_g.reshape(1, D), ln1_b.reshape(1, D))

    gate_spec = pltpu.PrefetchScalarGridSpec(
        num_scalar_prefetch=1,
        grid=(B,),
        in_specs=[pl.BlockSpec((S, D), lambda b, idx: (b, 0)),
                  pl.BlockSpec((1, E, D), lambda b, idx: (idx[b], 0, 0)),
                  pl.BlockSpec((1, 1, E), lambda b, idx: (idx[b], 0, 0))],
        out_specs=[pl.BlockSpec((1, 1, S), lambda b, idx: (b, 0, 0)),
                   pl.BlockSpec((1, 1, S), lambda b, idx: (b, 0, 0))],
    )
    gate, gval = pl.pallas_call(
        _gate_body,
        grid_spec=gate_spec,
        out_shape=[jax.ShapeDtypeStruct((B, 1, S), jnp.int32),
                   jax.ShapeDtypeStruct((B, 1, S), jnp.float32)],
    )(idxes, x, gate_W, gate_b.reshape(ND, 1, E))

    gate_flat = gate.reshape(TROWS, 128)
    dst, bex, src = pl.pallas_call(
        _route_body,
        grid=(NB + 1,),
        in_specs=[pl.BlockSpec((TROWS, 128), lambda j: (0, 0))],
        out_specs=[pl.BlockSpec((TROWS, 128), lambda j: (0, 0)),
                   pl.BlockSpec((1, NB), lambda j: (0, 0)),
                   pl.BlockSpec((1, 1, TBS),
                                lambda j: (jnp.maximum(j - 1, 0), 0, 0))],
        out_shape=[jax.ShapeDtypeStruct((TROWS, 128), jnp.int32),
                   jax.ShapeDtypeStruct((1, NB), jnp.int32),
                   jax.ShapeDtypeStruct((NB, 1, TBS), jnp.int32)],
        scratch_shapes=[pltpu.VMEM((TROWS, 128), jnp.int32)],
    )(gate_flat)

    xs = _sc_gather(x, src.reshape(PAD_T), PAD_T)

    expert_spec = pltpu.PrefetchScalarGridSpec(
        num_scalar_prefetch=1,
        grid=(NB,),
        in_specs=[pl.BlockSpec((TBS, D), lambda i, bx: (i, 0)),
                  pl.BlockSpec((1, INTER, D), lambda i, bx: (bx[i], 0, 0)),
                  pl.BlockSpec((1, 1, INTER), lambda i, bx: (bx[i], 0, 0)),
                  pl.BlockSpec((1, D, INTER), lambda i, bx: (bx[i], 0, 0))],
        out_specs=pl.BlockSpec((TBS, D), lambda i, bx: (i, 0)),
    )
    ys = pl.pallas_call(
        _expert_body,
        grid_spec=expert_spec,
        out_shape=jax.ShapeDtypeStruct((PAD_T, D), jnp.float32),
    )(bex.reshape(NB), xs, exp1_W, exp1_b.reshape(E, 1, INTER), exp2_W)

    ye = _sc_gather(ys, dst.reshape(T), T)

    gval_r = gval.reshape(NTB, 1, TB)
    out = pl.pallas_call(
        _final_body,
        grid=(NTB,),
        in_specs=[pl.BlockSpec((TB, D), lambda i: (i, 0)),
                  pl.BlockSpec((D, FFN), lambda i: (0, 0)),
                  pl.BlockSpec((1, FFN), lambda i: (0, 0)),
                  pl.BlockSpec((FFN, D), lambda i: (0, 0)),
                  pl.BlockSpec((1, D), lambda i: (0, 0)),
                  pl.BlockSpec((TB, D), lambda i: (i, 0)),
                  pl.BlockSpec((1, 1, TB), lambda i: (i, 0, 0)),
                  pl.BlockSpec((1, D), lambda i: (0, 0)),
                  pl.BlockSpec((1, D), lambda i: (0, 0))],
        out_specs=pl.BlockSpec((TB, D), lambda i: (i, 0)),
        out_shape=jax.ShapeDtypeStruct((T, D), jnp.float32),
    )(x, fc1_W.T, fc1_b.reshape(1, FFN), fc2_W.T, fc2_b.reshape(1, D),
      ye, gval_r, fln_g.reshape(1, D), fln_b.reshape(1, D))

    return out.reshape(B, S, D)
